# baseline (speedup 1.0000x reference)
import functools

import jax
import jax.numpy as jnp
from jax import lax
from jax.experimental import pallas as pl
from jax.experimental.pallas import tpu as pltpu
from jax.experimental.pallas import tpu_sc as plsc

B = 16384
D = 48
DP = 128
NV = 136
NC = 2
NS = 16
L = 16
NW = NC * NS
BPW = B // NW
CHUNK = 128
NCHUNK = BPW // CHUNK
HALF = B // 2
PPW = HALF // NW

_HI = lax.Precision.HIGHEST


def _table_body(role_ref, bat_ref, bowl_ref, w1_ref, b1_ref, w2_ref, b2_ref,
                feat_ref, out_ref, comb_ref):
    x = feat_ref[...]
    rr = jnp.clip(x[0:1, :], 0, 4)
    bt = jnp.clip(x[1:2, :], 0, 2)
    bw = jnp.clip(x[2:3, :], 0, 8)
    comb_ref[...] = (rr * 27 + bt * 9 + bw).reshape(B)
    f32 = jnp.float32
    ar = jnp.dot(role_ref[...], w1_ref[0:12, :], preferred_element_type=f32,
                 precision=_HI)
    ab = jnp.dot(bat_ref[...], w1_ref[12:20, :], preferred_element_type=f32,
                 precision=_HI)
    aw = jnp.dot(bowl_ref[...], w1_ref[20:28, :], preferred_element_type=f32,
                 precision=_HI)
    c = lax.broadcasted_iota(jnp.int32, (NV, 1), 0)
    r = c // 27
    b = (c // 9) % 3
    w = c % 9
    oh_r = (lax.broadcasted_iota(jnp.int32, (NV, 5), 1) == r).astype(f32)
    oh_b = (lax.broadcasted_iota(jnp.int32, (NV, 3), 1) == b).astype(f32)
    oh_w = (lax.broadcasted_iota(jnp.int32, (NV, 9), 1) == w).astype(f32)
    h = (jnp.dot(oh_r, ar, preferred_element_type=f32, precision=_HI)
         + jnp.dot(oh_b, ab, preferred_element_type=f32, precision=_HI)
         + jnp.dot(oh_w, aw, preferred_element_type=f32, precision=_HI)
         + b1_ref[...])
    h = jnp.maximum(h, 0.0)
    res = jnp.dot(h, w2_ref[...], preferred_element_type=f32,
                  precision=_HI) + b2_ref[...]
    out_ref[...] = jnp.concatenate(
        [res, jnp.zeros((NV, DP - D), dtype=f32)], axis=1)


def _build_table(role_table, batting_table, bowling_table, W1, b1, W2, b2,
                 feat_t):
    return pl.pallas_call(
        _table_body,
        out_shape=(jax.ShapeDtypeStruct((NV, DP), jnp.float32),
                   jax.ShapeDtypeStruct((B,), jnp.int32)),
    )(role_table, batting_table, bowling_table, W1,
      b1.reshape(1, D), W2, b2.reshape(1, D), feat_t)


def _sc_body(table_hbm, comb_hbm, out_hbm, idx_v, rows_v, packed_v, table_sh,
             sem, sem2):
    wid = lax.axis_index("s") * NC + lax.axis_index("c")
    pbase = wid * PPW
    c1 = pltpu.async_copy(comb_hbm.at[pl.ds(pbase, PPW)],
                          idx_v.at[pl.ds(0, PPW)], sem)
    c2 = pltpu.async_copy(comb_hbm.at[pl.ds(HALF + pbase, PPW)],
                          idx_v.at[pl.ds(PPW, PPW)], sem)

    @pl.when(lax.axis_index("s") == 0)
    def _():
        pltpu.async_copy(table_hbm, table_sh, sem2).wait()

    c1.wait()
    c2.wait()
    plsc.subcore_barrier()
    copies = [
        pltpu.async_copy(table_sh.at[idx_v.at[pl.ds(j * CHUNK, CHUNK)]],
                         rows_v.at[pl.ds(j * CHUNK, CHUNK)], sem)
        for j in range(NCHUNK)
    ]
    nhalf = NCHUNK // 2
    writes = []
    for j in range(nhalf):
        copies[j].wait()
        copies[j + nhalf].wait()
        for p in range(j * CHUNK, (j + 1) * CHUNK):
            for k in (0, 16, 32):
                packed_v[p, pl.ds(k, L)] = rows_v[p, pl.ds(k, L)]
                packed_v[p, pl.ds(D + k, L)] = rows_v[PPW + p, pl.ds(k, L)]
        writes.append(
            pltpu.async_copy(packed_v.at[pl.ds(j * CHUNK, CHUNK)],
                             out_hbm.at[pl.ds(pbase + j * CHUNK, CHUNK)],
                             sem2))
    for w in writes:
        w.wait()


@functools.lru_cache(maxsize=1)
def _sc_gather_fn():
    mesh = plsc.VectorSubcoreMesh(core_axis_name="c", subcore_axis_name="s",
                                  num_cores=NC, num_subcores=NS)
    return pl.kernel(
        _sc_body,
        out_type=jax.ShapeDtypeStruct((HALF, DP), jnp.float32),
        mesh=mesh,
        scratch_types=[
            pltpu.VMEM((BPW,), jnp.int32),
            pltpu.VMEM((BPW, DP), jnp.float32),
            pltpu.VMEM((PPW, DP), jnp.float32),
            pltpu.VMEM_SHARED((NV, DP), jnp.float32),
            pltpu.SemaphoreType.DMA,
            pltpu.SemaphoreType.DMA,
        ],
        compiler_params=pltpu.CompilerParams(use_tc_tiling_on_sc=True),
    )


_RB = 4096
_NB = HALF // _RB


def _slice_body(in_ref, out_ref):
    h = pl.program_id(1)
    lo = in_ref[:, 0:D].T
    hi = in_ref[:, D:2 * D].T
    out_ref[...] = jnp.where(h == 0, lo, hi)


def _slice_cols(packed):
    return pl.pallas_call(
        _slice_body,
        grid=(_NB, 2),
        in_specs=[pl.BlockSpec((_RB, DP), lambda i, h: (i, 0))],
        out_specs=pl.BlockSpec((D, _RB), lambda i, h: (0, h * _NB + i)),
        out_shape=jax.ShapeDtypeStruct((D, B), jnp.float32),
    )(packed)


def kernel(squad_features, role_table, batting_table, bowling_table,
           W1, b1, W2, b2):
    table, comb = _build_table(role_table, batting_table, bowling_table,
                               W1, b1, W2, b2, squad_features.T)
    padded = _sc_gather_fn()(table, comb)
    return _slice_cols(padded).T


# speedup vs baseline: 1.1979x; 1.1979x over previous
import functools

import jax
import jax.numpy as jnp
from jax import lax
from jax.experimental import pallas as pl
from jax.experimental.pallas import tpu as pltpu
from jax.experimental.pallas import tpu_sc as plsc

B = 16384
D = 48
DP = 128
NV = 136
NC = 2
NS = 16
L = 16
NW = NC * NS
BPW = B // NW
CHUNK = 64
NCHUNK = BPW // CHUNK

_HI = lax.Precision.HIGHEST


def _table_body(role_ref, bat_ref, bowl_ref, w1_ref, b1_ref, w2_ref, b2_ref,
                feat_ref, out_ref, comb_ref):
    x = feat_ref[...]
    rr = jnp.clip(x[0:1, :], 0, 4)
    bt = jnp.clip(x[1:2, :], 0, 2)
    bw = jnp.clip(x[2:3, :], 0, 8)
    comb_ref[...] = (rr * 27 + bt * 9 + bw).reshape(B)
    f32 = jnp.float32
    ar = jnp.dot(role_ref[...], w1_ref[0:12, :], preferred_element_type=f32,
                 precision=_HI)
    ab = jnp.dot(bat_ref[...], w1_ref[12:20, :], preferred_element_type=f32,
                 precision=_HI)
    aw = jnp.dot(bowl_ref[...], w1_ref[20:28, :], preferred_element_type=f32,
                 precision=_HI)
    c = lax.broadcasted_iota(jnp.int32, (NV, 1), 0)
    r = c // 27
    b = (c // 9) % 3
    w = c % 9
    oh_r = (lax.broadcasted_iota(jnp.int32, (NV, 5), 1) == r).astype(f32)
    oh_b = (lax.broadcasted_iota(jnp.int32, (NV, 3), 1) == b).astype(f32)
    oh_w = (lax.broadcasted_iota(jnp.int32, (NV, 9), 1) == w).astype(f32)
    h = (jnp.dot(oh_r, ar, preferred_element_type=f32, precision=_HI)
         + jnp.dot(oh_b, ab, preferred_element_type=f32, precision=_HI)
         + jnp.dot(oh_w, aw, preferred_element_type=f32, precision=_HI)
         + b1_ref[...])
    h = jnp.maximum(h, 0.0)
    res = jnp.dot(h, w2_ref[...], preferred_element_type=f32,
                  precision=_HI) + b2_ref[...]
    out_ref[...] = jnp.concatenate(
        [res, jnp.zeros((NV, DP - D), dtype=f32)], axis=1)


def _build_table(role_table, batting_table, bowling_table, W1, b1, W2, b2,
                 feat_t):
    return pl.pallas_call(
        _table_body,
        out_shape=(jax.ShapeDtypeStruct((NV, DP), jnp.float32),
                   jax.ShapeDtypeStruct((B,), jnp.int32)),
    )(role_table, batting_table, bowling_table, W1,
      b1.reshape(1, D), W2, b2.reshape(1, D), feat_t)


def _sc_body(table_hbm, comb_hbm, out_hbm, idx_v, rows_v, table_sh, sem, sem2):
    wid = lax.axis_index("s") * NC + lax.axis_index("c")
    base = wid * BPW
    c1 = pltpu.async_copy(comb_hbm.at[pl.ds(base, BPW)], idx_v, sem)

    @pl.when(lax.axis_index("s") == 0)
    def _():
        pltpu.async_copy(table_hbm, table_sh, sem2).wait()

    c1.wait()
    plsc.subcore_barrier()
    copies = [
        pltpu.async_copy(table_sh.at[idx_v.at[pl.ds(j * CHUNK, CHUNK)]],
                         rows_v.at[pl.ds(j * CHUNK, CHUNK)], sem)
        for j in range(NCHUNK)
    ]
    writes = []
    for j in range(NCHUNK):
        copies[j].wait()
        writes.append(
            pltpu.async_copy(rows_v.at[pl.ds(j * CHUNK, CHUNK)],
                             out_hbm.at[pl.ds(base + j * CHUNK, CHUNK)],
                             sem2))
    for w in writes:
        w.wait()


@functools.lru_cache(maxsize=1)
def _sc_gather_fn():
    mesh = plsc.VectorSubcoreMesh(core_axis_name="c", subcore_axis_name="s",
                                  num_cores=NC, num_subcores=NS)
    return pl.kernel(
        _sc_body,
        out_type=jax.ShapeDtypeStruct((B, DP), jnp.float32),
        mesh=mesh,
        scratch_types=[
            pltpu.VMEM((BPW,), jnp.int32),
            pltpu.VMEM((BPW, DP), jnp.float32),
            pltpu.VMEM_SHARED((NV, DP), jnp.float32),
            pltpu.SemaphoreType.DMA,
            pltpu.SemaphoreType.DMA,
        ],
        compiler_params=pltpu.CompilerParams(use_tc_tiling_on_sc=True),
    )


_RB = 8192


def _slice_body(in_ref, out_ref):
    out_ref[...] = in_ref[:, 0:D].T


def _slice_cols(padded):
    return pl.pallas_call(
        _slice_body,
        grid=(B // _RB,),
        in_specs=[pl.BlockSpec((_RB, DP), lambda i: (i, 0))],
        out_specs=pl.BlockSpec((D, _RB), lambda i: (0, i)),
        out_shape=jax.ShapeDtypeStruct((D, B), jnp.float32),
    )(padded)


def kernel(squad_features, role_table, batting_table, bowling_table,
           W1, b1, W2, b2):
    table, comb = _build_table(role_table, batting_table, bowling_table,
                               W1, b1, W2, b2, squad_features.T)
    padded = _sc_gather_fn()(table, comb)
    return _slice_cols(padded).T
